# Initial kernel scaffold; baseline (speedup 1.0000x reference)
import jax
import jax.numpy as jnp
from jax.experimental import pallas as pl


def kernel(x, edge_index, edge_attr, params):
    raise NotImplementedError("write your pallas kernel here")


# speedup vs baseline: 2.4307x; 2.4307x over previous
import jax
import jax.numpy as jnp
from jax import lax
from jax.experimental import pallas as pl
from jax.experimental.pallas import tpu as pltpu
from jax.experimental.pallas import tpu_sc as plsc

N = 10000
E = 160000
K = 15
D = 3
EPS = 1e-15

NC = 2
NS = 16
NH = N // NC
AGG_R = NH + 8
B = 20
C = 96000
SLOTS = NC * C
EPT = C // NS
NB = EPT // B
DR = 100
NCHUNK = NH // DR

FIN = [50, 75, 100, 75]
FOUT = [75, 100, 75, 50]
FP = [80, 112, 80, 64]

MBLK = 1000
NBLK = N // MBLK
EBLK = 2000
GBLK = SLOTS // EBLK


def _gauss_body(ea_ref, c2_ref, c1_ref, *outs):
    ea = ea_ref[...]
    q = (jnp.dot(ea * ea, c2_ref[...], preferred_element_type=jnp.float32)
         + jnp.dot(ea, c1_ref[...], preferred_element_type=jnp.float32))
    g = jnp.exp(q)
    for i, o in enumerate(outs):
        o[...] = g[:, 16 * i:16 * (i + 1)]


def _gauss_all(ea8, c2, c1):
    return pl.pallas_call(
        _gauss_body,
        grid=(GBLK,),
        in_specs=[
            pl.BlockSpec((EBLK, 8), lambda i: (i, 0)),
            pl.BlockSpec((8, 64), lambda i: (0, 0)),
            pl.BlockSpec((8, 64), lambda i: (0, 0)),
        ],
        out_specs=[pl.BlockSpec((EBLK, 16), lambda i: (i, 0))] * 4,
        out_shape=[jax.ShapeDtypeStruct((SLOTS, 16), jnp.float32)] * 4,
    )(ea8, c2, c1)


def _affine(p, gb):
    s = jnp.sum(p, axis=0)
    mean = s[0:1, :] * (1.0 / N)
    var = s[1:2, :] * (1.0 / N) - mean * mean
    a = gb[0:1, :] * lax.rsqrt(var + 1e-5)
    b = gb[1:2, :] - mean * a
    return a, b


def _mm_body(h_ref, w_ref, o_ref):
    o_ref[...] = jnp.dot(h_ref[...], w_ref[...],
                         preferred_element_type=jnp.float32)


def _mm_bn_body(h_ref, p_ref, gb_ref, w_ref, o_ref):
    a, b = _affine(p_ref[...], gb_ref[...])
    o_ref[...] = jnp.dot(h_ref[...] * a + b, w_ref[...],
                         preferred_element_type=jnp.float32)


def _matmul(h, w, p=None, gb=None):
    fin, fout = w.shape
    if p is None:
        return pl.pallas_call(
            _mm_body,
            grid=(NBLK,),
            in_specs=[
                pl.BlockSpec((MBLK, fin), lambda i: (i, 0)),
                pl.BlockSpec((fin, fout), lambda i: (0, 0)),
            ],
            out_specs=pl.BlockSpec((MBLK, fout), lambda i: (i, 0)),
            out_shape=jax.ShapeDtypeStruct((N, fout), jnp.float32),
        )(h, w)
    return pl.pallas_call(
        _mm_bn_body,
        grid=(NBLK,),
        in_specs=[
            pl.BlockSpec((MBLK, fin), lambda i: (i, 0)),
            pl.BlockSpec((NBLK, 8, fin), lambda i: (0, 0, 0)),
            pl.BlockSpec((8, fin), lambda i: (0, 0)),
            pl.BlockSpec((fin, fout), lambda i: (0, 0)),
        ],
        out_specs=pl.BlockSpec((MBLK, fout), lambda i: (i, 0)),
        out_shape=jax.ShapeDtypeStruct((N, fout), jnp.float32),
    )(h, p, gb, w)


def _elu(x):
    return jnp.where(x > 0, x, jnp.exp(jnp.minimum(x, 0.0)) - 1.0)


def _partials(h, fp):
    s1 = jnp.sum(h, axis=0, keepdims=True)
    s2 = jnp.sum(h * h, axis=0, keepdims=True)
    return jnp.concatenate([s1, s2, jnp.zeros((6, fp), jnp.float32)],
                           axis=0)[None]


def _comb1_body(agg_ref, cnt_ref, x_ref, root_ref, bias_ref,
                h_ref, p_ref, inv_ref):
    cnt16 = cnt_ref[...]
    lane0 = (lax.broadcasted_iota(jnp.int32, (1, 16), 1) == 0)
    cnt = jnp.sum(jnp.where(lane0, cnt16, 0.0), axis=1)
    inv = (1.0 / jnp.maximum(cnt, 1.0))[:, None]
    raw = (agg_ref[...] * inv
           + jnp.dot(x_ref[...], root_ref[...],
                     preferred_element_type=jnp.float32) + bias_ref[0:1, :])
    h = _elu(raw)
    h_ref[...] = h
    p_ref[...] = _partials(h, h.shape[-1])
    inv_ref[...] = jnp.broadcast_to(inv, (inv.shape[0], 16))


def _combine1(agg, cnt, x, root, bias):
    fp = FP[0]
    return pl.pallas_call(
        _comb1_body,
        grid=(NBLK,),
        in_specs=[
            pl.BlockSpec((MBLK, fp), lambda i: (i, 0)),
            pl.BlockSpec((MBLK, 16), lambda i: (i, 0)),
            pl.BlockSpec((MBLK, 50), lambda i: (i, 0)),
            pl.BlockSpec((50, fp), lambda i: (0, 0)),
            pl.BlockSpec((8, fp), lambda i: (0, 0)),
        ],
        out_specs=[
            pl.BlockSpec((MBLK, fp), lambda i: (i, 0)),
            pl.BlockSpec((1, 8, fp), lambda i: (i, 0, 0)),
            pl.BlockSpec((MBLK, 16), lambda i: (i, 0)),
        ],
        out_shape=[
            jax.ShapeDtypeStruct((N, fp), jnp.float32),
            jax.ShapeDtypeStruct((NBLK, 8, fp), jnp.float32),
            jax.ShapeDtypeStruct((N, 16), jnp.float32),
        ],
    )(agg, cnt, x, root, bias)


def _comb_body(agg_ref, invc_ref, h_ref, p_ref, gb_ref, root_ref, bias_ref,
               ho_ref, po_ref, *, last):
    inv = jnp.max(invc_ref[...], axis=1, keepdims=True)
    a, b = _affine(p_ref[...], gb_ref[...])
    hb = h_ref[...] * a + b
    raw = (agg_ref[...] * inv
           + jnp.dot(hb, root_ref[...], preferred_element_type=jnp.float32)
           + bias_ref[0:1, :])
    if last:
        ho_ref[...] = raw
    else:
        h = _elu(raw)
        ho_ref[...] = h
        po_ref[...] = _partials(h, h.shape[-1])


def _combine(agg, invc, h, p, gb, root, bias, last):
    fin, fp = root.shape
    if last:
        body = lambda a, i, hh, pp, g, r, bb, ho: _comb_body(
            a, i, hh, pp, g, r, bb, ho, None, last=True)
    else:
        body = lambda a, i, hh, pp, g, r, bb, ho, po: _comb_body(
            a, i, hh, pp, g, r, bb, ho, po, last=False)
    out_specs = [pl.BlockSpec((MBLK, fp), lambda i: (i, 0))]
    out_shape = [jax.ShapeDtypeStruct((N, fp), jnp.float32)]
    if not last:
        out_specs.append(pl.BlockSpec((1, 8, fp), lambda i: (i, 0, 0)))
        out_shape.append(jax.ShapeDtypeStruct((NBLK, 8, fp), jnp.float32))
    return pl.pallas_call(
        body,
        grid=(NBLK,),
        in_specs=[
            pl.BlockSpec((MBLK, fp), lambda i: (i, 0)),
            pl.BlockSpec((MBLK, 16), lambda i: (i, 0)),
            pl.BlockSpec((MBLK, fin), lambda i: (i, 0)),
            pl.BlockSpec((NBLK, 8, fin), lambda i: (0, 0, 0)),
            pl.BlockSpec((8, fin), lambda i: (0, 0)),
            pl.BlockSpec((fin, fp), lambda i: (0, 0)),
            pl.BlockSpec((8, fp), lambda i: (0, 0)),
        ],
        out_specs=out_specs,
        out_shape=out_shape,
    )(agg, invc, h, p, gb, root, bias)


def _make_sc_msg(fp, with_count):
    kfp = K * fp
    nf = fp // 16
    mesh = plsc.VectorSubcoreMesh(core_axis_name="c", subcore_axis_name="s")

    out_type = [jax.ShapeDtypeStruct((N, fp), jnp.float32)]
    scratch = [
        pltpu.VMEM((NB, B), jnp.int32),
        pltpu.VMEM((NB, B), jnp.int32),
        pltpu.VMEM((2, B, kfp), jnp.float32),
        pltpu.VMEM((2, B, 16), jnp.float32),
        pltpu.VMEM((B, fp), jnp.float32),
        pltpu.VMEM((DR, fp), jnp.float32),
        pltpu.VMEM_SHARED((AGG_R, fp), jnp.float32),
        pltpu.SemaphoreType.DMA,
        pltpu.SemaphoreType.DMA,
        pltpu.SemaphoreType.DMA,
        pltpu.SemaphoreType.DMA,
    ]
    if with_count:
        out_type.append(jax.ShapeDtypeStruct((N, 16), jnp.float32))
        scratch += [
            pltpu.VMEM((B, 16), jnp.float32),
            pltpu.VMEM((DR, 16), jnp.float32),
            pltpu.VMEM_SHARED((AGG_R, 16), jnp.float32),
        ]

    def body(outn, gauss, srcg, dstg, *rest):
        if with_count:
            (out, cnt_out, s_v, d_v, rows_v, gs_v, msg_v, z_v, agg_sh,
             sr0, sr1, sg0, sg1, cmsg_v, zc_v, cnt_sh) = rest
        else:
            (out, s_v, d_v, rows_v, gs_v, msg_v, z_v, agg_sh,
             sr0, sr1, sg0, sg1) = rest
        c = lax.axis_index("c")
        s = lax.axis_index("s")
        w = c * NS + s
        rb = w * NB
        eb = w * EPT

        pltpu.sync_copy(srcg.at[pl.ds(rb, NB)], s_v)
        pltpu.sync_copy(dstg.at[pl.ds(rb, NB)], d_v)

        zero16 = jnp.zeros((16,), jnp.float32)

        def zz(i, _):
            for f in range(nf):
                z_v[i, pl.ds(16 * f, 16)] = zero16
            if with_count:
                zc_v[i, pl.ds(0, 16)] = zero16
            return 0
        lax.fori_loop(0, DR, zz, 0)
        for i in range(4):
            q = s + 16 * i

            @pl.when(q < NCHUNK)
            def _():
                pltpu.sync_copy(z_v, agg_sh.at[pl.ds(q * DR, DR)])
                if with_count:
                    pltpu.sync_copy(zc_v, cnt_sh.at[pl.ds(q * DR, DR)])
        @pl.when(s == 0)
        def _():
            pltpu.sync_copy(z_v.at[pl.ds(0, 8)], agg_sh.at[pl.ds(NH, 8)])
            if with_count:
                pltpu.sync_copy(zc_v.at[pl.ds(0, 8)],
                                cnt_sh.at[pl.ds(NH, 8)])
        if with_count:
            one0 = jnp.where(
                lax.iota(jnp.int32, 16) == 0, 1.0, 0.0).astype(jnp.float32)

            def co(i, _):
                cmsg_v[i, pl.ds(0, 16)] = one0
                return 0
            lax.fori_loop(0, B, co, 0)
        plsc.subcore_barrier()

        sem_r = (sr0, sr1)
        sem_g = (sg0, sg1)

        def start(j, b):
            pltpu.async_copy(outn.at[s_v.at[j]], rows_v.at[b], sem_r[b])
            pltpu.async_copy(gauss.at[pl.ds(eb + j * B, B)],
                             gs_v.at[b], sem_g[b])

        def wait(j, b):
            pltpu.make_async_copy(
                outn.at[s_v.at[j]], rows_v.at[b], sem_r[b]).wait()
            pltpu.make_async_copy(
                gauss.at[pl.ds(eb + j * B, B)], gs_v.at[b], sem_g[b]).wait()

        def compute_scatter(j, b):
            def edge(i, _):
                accs = [zero16] * nf
                gvec = gs_v[b, i, pl.ds(0, 16)]
                for k in range(K):
                    wk = gvec[k]
                    for f in range(nf):
                        accs[f] = accs[f] + wk * rows_v[
                            b, i, pl.ds(k * fp + 16 * f, 16)]
                for f in range(nf):
                    msg_v[i, pl.ds(16 * f, 16)] = accs[f]
                return 0
            lax.fori_loop(0, B, edge, 0)
            pltpu.sync_copy(msg_v, agg_sh.at[d_v.at[j]], add=True)
            if with_count:
                pltpu.sync_copy(cmsg_v, cnt_sh.at[d_v.at[j]], add=True)

        start(0, 0)

        def step(jj, _):
            j0 = 2 * jj
            wait(j0, 0)
            start(j0 + 1, 1)
            compute_scatter(j0, 0)
            wait(j0 + 1, 1)

            @pl.when(jj < NB // 2 - 1)
            def _():
                start(j0 + 2, 0)
            compute_scatter(j0 + 1, 1)
            return 0
        lax.fori_loop(0, NB // 2, step, 0)

        plsc.subcore_barrier()
        for i in range(4):
            q = s + 16 * i

            @pl.when(q < NCHUNK)
            def _():
                sl = pl.ds(q * DR, DR)
                osl = pl.ds(c * NH + q * DR, DR)
                pltpu.sync_copy(agg_sh.at[sl], out.at[osl])
                if with_count:
                    pltpu.sync_copy(cnt_sh.at[sl], cnt_out.at[osl])

    return pl.kernel(
        body, out_type=out_type, mesh=mesh, scratch_types=scratch,
        compiler_params=pltpu.CompilerParams(use_tc_tiling_on_sc=False))


def _pad_cols(w, out_w):
    return jnp.pad(w, ((0, 0), (0, out_w - w.shape[1])))


def _prep_g(g, f_out, fp, fin_pad=None):
    f_in = g.shape[0]
    g3 = g.reshape(f_in, K, f_out)
    g3 = jnp.pad(g3, ((0, 0), (0, 0), (0, fp - f_out)))
    g2 = g3.reshape(f_in, K * fp)
    if fin_pad is not None and fin_pad != f_in:
        g2 = jnp.pad(g2, ((0, fin_pad - f_in), (0, 0)))
    return g2


def _prep_root(r, fp, fin_pad=None):
    r2 = _pad_cols(r, fp)
    if fin_pad is not None and fin_pad != r.shape[0]:
        r2 = jnp.pad(r2, ((0, fin_pad - r.shape[0]), (0, 0)))
    return r2


def _prep_vec8(v, fp, row=0):
    out = jnp.zeros((8, fp), jnp.float32)
    return out.at[row, :v.shape[0]].set(v)


def _prep_gb(gamma, beta, fp):
    out = jnp.zeros((8, fp), jnp.float32)
    out = out.at[0, :gamma.shape[0]].set(gamma)
    return out.at[1, :beta.shape[0]].set(beta)


def _gauss_coeffs(params):
    c2 = jnp.zeros((8, 64), jnp.float32)
    c1 = jnp.zeros((8, 64), jnp.float32)
    for l, name in enumerate(['conv1', 'conv2', 'conv3', 'conv4']):
        p = params[name]
        inv = -0.5 / (EPS + p['sigma'] ** 2)
        c2 = c2.at[:D, 16 * l:16 * l + K].set(inv.T)
        c1 = c1.at[:D, 16 * l:16 * l + K].set((-2.0 * inv * p['mu']).T)
        c1 = c1.at[3, 16 * l:16 * l + K].set(
            jnp.sum(inv * p['mu'] ** 2, axis=1))
    return c2, c1


def kernel(x, edge_index, edge_attr, params):
    src = edge_index[0].astype(jnp.int32)
    dst = edge_index[1].astype(jnp.int32)

    hi = dst >= NH
    r1 = jnp.cumsum(hi.astype(jnp.int32))
    r0 = jnp.cumsum(1 - hi.astype(jnp.int32))
    slot = jnp.where(hi, C + r1 - 1, r0 - 1)
    src_p = jnp.zeros((SLOTS,), jnp.int32).at[slot].set(src)
    dst_p = jnp.full((SLOTS,), NH, jnp.int32).at[slot].set(
        dst - jnp.where(hi, NH, 0))
    ea8 = jnp.concatenate(
        [edge_attr, jnp.ones((E, 1), jnp.float32),
         jnp.zeros((E, 4), jnp.float32)], axis=1)
    ea_p = jnp.zeros((SLOTS, 8), jnp.float32).at[slot].set(ea8)

    src2 = src_p.reshape(SLOTS // B, B)
    dst2 = dst_p.reshape(SLOTS // B, B)

    c2, c1 = _gauss_coeffs(params)
    gs = _gauss_all(ea_p, c2, c1)

    sc_msg = [_make_sc_msg(FP[0], True)] + [
        _make_sc_msg(FP[l], False) for l in (1, 2, 3)]

    p1 = params['conv1']

    outn = _matmul(x, _prep_g(p1['g'], FOUT[0], FP[0]))
    agg, cnt = sc_msg[0](outn, gs[0], src2, dst2)
    h, part, invc = _combine1(agg, cnt, x,
                              _prep_root(p1['root'], FP[0]),
                              _prep_vec8(p1['bias'], FP[0]))

    convs = [params['conv2'], params['conv3'], params['conv4']]
    bns = [params['bn1'], params['bn2'], params['bn3']]
    out = None
    for l in (1, 2, 3):
        p = convs[l - 1]
        gb = _prep_gb(bns[l - 1]['gamma'], bns[l - 1]['beta'], FP[l - 1])
        outn = _matmul(h, _prep_g(p['g'], FOUT[l], FP[l], FP[l - 1]),
                       part, gb)
        agg = sc_msg[l](outn, gs[l], src2, dst2)[0]
        res = _combine(agg, invc, h, part, gb,
                       _prep_root(p['root'], FP[l], FP[l - 1]),
                       _prep_vec8(p['bias'], FP[l]), last=(l == 3))
        if l == 3:
            out = res[0]
        else:
            h, part = res

    return out[:, :FOUT[3]]


# speedup vs baseline: 2.4312x; 1.0002x over previous
import jax
import jax.numpy as jnp
from jax import lax
from jax.experimental import pallas as pl
from jax.experimental.pallas import tpu as pltpu
from jax.experimental.pallas import tpu_sc as plsc

N = 10000
E = 160000
K = 15
D = 3
EPS = 1e-15

NC = 2
NS = 16
NH = N // NC
AGG_R = NH + 8
B = 20
C = 96000
SLOTS = NC * C
EPT = C // NS
NB = EPT // B
DR = 100
NCHUNK = NH // DR

FIN = [50, 75, 100, 75]
FOUT = [75, 100, 75, 50]
FP = [80, 112, 80, 64]

MBLK = 1000
NBLK = N // MBLK
EBLK = 2000
GBLK = SLOTS // EBLK


def _gauss_body(ea_ref, c2_ref, c1_ref, *outs):
    ea = ea_ref[...]
    q = (jnp.dot(ea * ea, c2_ref[...], preferred_element_type=jnp.float32)
         + jnp.dot(ea, c1_ref[...], preferred_element_type=jnp.float32))
    g = jnp.exp(q)
    for i, o in enumerate(outs):
        o[...] = g[:, 16 * i:16 * (i + 1)]


def _gauss_all(ea8, c2, c1):
    return pl.pallas_call(
        _gauss_body,
        grid=(GBLK,),
        in_specs=[
            pl.BlockSpec((EBLK, 8), lambda i: (i, 0)),
            pl.BlockSpec((8, 64), lambda i: (0, 0)),
            pl.BlockSpec((8, 64), lambda i: (0, 0)),
        ],
        out_specs=[pl.BlockSpec((EBLK, 16), lambda i: (i, 0))] * 4,
        out_shape=[jax.ShapeDtypeStruct((SLOTS, 16), jnp.float32)] * 4,
    )(ea8, c2, c1)


def _affine(p, gb):
    s = jnp.sum(p, axis=0)
    mean = s[0:1, :] * (1.0 / N)
    var = s[1:2, :] * (1.0 / N) - mean * mean
    a = gb[0:1, :] * lax.rsqrt(var + 1e-5)
    b = gb[1:2, :] - mean * a
    return a, b


def _mm_body(h_ref, w_ref, o_ref):
    o_ref[...] = jnp.dot(h_ref[...], w_ref[...],
                         preferred_element_type=jnp.float32)


def _mm_bn_body(h_ref, p_ref, gb_ref, w_ref, o_ref):
    a, b = _affine(p_ref[...], gb_ref[...])
    o_ref[...] = jnp.dot(h_ref[...] * a + b, w_ref[...],
                         preferred_element_type=jnp.float32)


def _matmul(h, w, p=None, gb=None):
    fin, fout = w.shape
    if p is None:
        return pl.pallas_call(
            _mm_body,
            grid=(NBLK,),
            in_specs=[
                pl.BlockSpec((MBLK, fin), lambda i: (i, 0)),
                pl.BlockSpec((fin, fout), lambda i: (0, 0)),
            ],
            out_specs=pl.BlockSpec((MBLK, fout), lambda i: (i, 0)),
            out_shape=jax.ShapeDtypeStruct((N, fout), jnp.float32),
        )(h, w)
    return pl.pallas_call(
        _mm_bn_body,
        grid=(NBLK,),
        in_specs=[
            pl.BlockSpec((MBLK, fin), lambda i: (i, 0)),
            pl.BlockSpec((NBLK, 8, fin), lambda i: (0, 0, 0)),
            pl.BlockSpec((8, fin), lambda i: (0, 0)),
            pl.BlockSpec((fin, fout), lambda i: (0, 0)),
        ],
        out_specs=pl.BlockSpec((MBLK, fout), lambda i: (i, 0)),
        out_shape=jax.ShapeDtypeStruct((N, fout), jnp.float32),
    )(h, p, gb, w)


def _elu(x):
    return jnp.where(x > 0, x, jnp.exp(jnp.minimum(x, 0.0)) - 1.0)


def _partials(h, fp):
    s1 = jnp.sum(h, axis=0, keepdims=True)
    s2 = jnp.sum(h * h, axis=0, keepdims=True)
    return jnp.concatenate([s1, s2, jnp.zeros((6, fp), jnp.float32)],
                           axis=0)[None]


def _comb1_body(agg_ref, cnt_ref, x_ref, root_ref, bias_ref,
                h_ref, p_ref, inv_ref):
    cnt16 = cnt_ref[...]
    lane0 = (lax.broadcasted_iota(jnp.int32, (1, 16), 1) == 0)
    cnt = jnp.sum(jnp.where(lane0, cnt16, 0.0), axis=1)
    inv = (1.0 / jnp.maximum(cnt, 1.0))[:, None]
    raw = (agg_ref[...] * inv
           + jnp.dot(x_ref[...], root_ref[...],
                     preferred_element_type=jnp.float32) + bias_ref[0:1, :])
    h = _elu(raw)
    h_ref[...] = h
    p_ref[...] = _partials(h, h.shape[-1])
    inv_ref[...] = jnp.broadcast_to(inv, (inv.shape[0], 16))


def _combine1(agg, cnt, x, root, bias):
    fp = FP[0]
    return pl.pallas_call(
        _comb1_body,
        grid=(NBLK,),
        in_specs=[
            pl.BlockSpec((MBLK, fp), lambda i: (i, 0)),
            pl.BlockSpec((MBLK, 16), lambda i: (i, 0)),
            pl.BlockSpec((MBLK, 50), lambda i: (i, 0)),
            pl.BlockSpec((50, fp), lambda i: (0, 0)),
            pl.BlockSpec((8, fp), lambda i: (0, 0)),
        ],
        out_specs=[
            pl.BlockSpec((MBLK, fp), lambda i: (i, 0)),
            pl.BlockSpec((1, 8, fp), lambda i: (i, 0, 0)),
            pl.BlockSpec((MBLK, 16), lambda i: (i, 0)),
        ],
        out_shape=[
            jax.ShapeDtypeStruct((N, fp), jnp.float32),
            jax.ShapeDtypeStruct((NBLK, 8, fp), jnp.float32),
            jax.ShapeDtypeStruct((N, 16), jnp.float32),
        ],
    )(agg, cnt, x, root, bias)


def _comb_body(agg_ref, invc_ref, h_ref, p_ref, gb_ref, root_ref, bias_ref,
               ho_ref, po_ref, *, last):
    inv = jnp.max(invc_ref[...], axis=1, keepdims=True)
    a, b = _affine(p_ref[...], gb_ref[...])
    hb = h_ref[...] * a + b
    raw = (agg_ref[...] * inv
           + jnp.dot(hb, root_ref[...], preferred_element_type=jnp.float32)
           + bias_ref[0:1, :])
    if last:
        ho_ref[...] = raw
    else:
        h = _elu(raw)
        ho_ref[...] = h
        po_ref[...] = _partials(h, h.shape[-1])


def _combine(agg, invc, h, p, gb, root, bias, last):
    fin, fp = root.shape
    if last:
        body = lambda a, i, hh, pp, g, r, bb, ho: _comb_body(
            a, i, hh, pp, g, r, bb, ho, None, last=True)
    else:
        body = lambda a, i, hh, pp, g, r, bb, ho, po: _comb_body(
            a, i, hh, pp, g, r, bb, ho, po, last=False)
    out_specs = [pl.BlockSpec((MBLK, fp), lambda i: (i, 0))]
    out_shape = [jax.ShapeDtypeStruct((N, fp), jnp.float32)]
    if not last:
        out_specs.append(pl.BlockSpec((1, 8, fp), lambda i: (i, 0, 0)))
        out_shape.append(jax.ShapeDtypeStruct((NBLK, 8, fp), jnp.float32))
    return pl.pallas_call(
        body,
        grid=(NBLK,),
        in_specs=[
            pl.BlockSpec((MBLK, fp), lambda i: (i, 0)),
            pl.BlockSpec((MBLK, 16), lambda i: (i, 0)),
            pl.BlockSpec((MBLK, fin), lambda i: (i, 0)),
            pl.BlockSpec((NBLK, 8, fin), lambda i: (0, 0, 0)),
            pl.BlockSpec((8, fin), lambda i: (0, 0)),
            pl.BlockSpec((fin, fp), lambda i: (0, 0)),
            pl.BlockSpec((8, fp), lambda i: (0, 0)),
        ],
        out_specs=out_specs,
        out_shape=out_shape,
    )(agg, invc, h, p, gb, root, bias)


def _make_sc_msg(fp, with_count):
    kfp = K * fp
    nf = fp // 16
    mesh = plsc.VectorSubcoreMesh(core_axis_name="c", subcore_axis_name="s")

    out_type = [jax.ShapeDtypeStruct((N, fp), jnp.float32)]
    scratch = [
        pltpu.VMEM((NB, B), jnp.int32),
        pltpu.VMEM((NB, B), jnp.int32),
        pltpu.VMEM((2, B, kfp), jnp.float32),
        pltpu.VMEM((2, B, 16), jnp.float32),
        pltpu.VMEM((B, fp), jnp.float32),
        pltpu.VMEM((DR, fp), jnp.float32),
        pltpu.VMEM_SHARED((AGG_R, fp), jnp.float32),
        pltpu.SemaphoreType.DMA,
        pltpu.SemaphoreType.DMA,
        pltpu.SemaphoreType.DMA,
        pltpu.SemaphoreType.DMA,
    ]
    if with_count:
        out_type.append(jax.ShapeDtypeStruct((N, 16), jnp.float32))
        scratch += [
            pltpu.VMEM((B, 16), jnp.float32),
            pltpu.VMEM((DR, 16), jnp.float32),
            pltpu.VMEM_SHARED((AGG_R, 16), jnp.float32),
        ]

    def body(outn, gauss, srcg, dstg, *rest):
        if with_count:
            (out, cnt_out, s_v, d_v, rows_v, gs_v, msg_v, z_v, agg_sh,
             sr0, sr1, sg0, sg1, cmsg_v, zc_v, cnt_sh) = rest
        else:
            (out, s_v, d_v, rows_v, gs_v, msg_v, z_v, agg_sh,
             sr0, sr1, sg0, sg1) = rest
        c = lax.axis_index("c")
        s = lax.axis_index("s")
        w = c * NS + s
        rb = w * NB
        eb = w * EPT

        pltpu.sync_copy(srcg.at[pl.ds(rb, NB)], s_v)
        pltpu.sync_copy(dstg.at[pl.ds(rb, NB)], d_v)

        zero16 = jnp.zeros((16,), jnp.float32)

        def zz(i, _):
            for f in range(nf):
                z_v[i, pl.ds(16 * f, 16)] = zero16
            if with_count:
                zc_v[i, pl.ds(0, 16)] = zero16
            return 0
        lax.fori_loop(0, DR, zz, 0)
        for i in range(4):
            q = s + 16 * i

            @pl.when(q < NCHUNK)
            def _():
                pltpu.sync_copy(z_v, agg_sh.at[pl.ds(q * DR, DR)])
                if with_count:
                    pltpu.sync_copy(zc_v, cnt_sh.at[pl.ds(q * DR, DR)])
        @pl.when(s == 0)
        def _():
            pltpu.sync_copy(z_v.at[pl.ds(0, 8)], agg_sh.at[pl.ds(NH, 8)])
            if with_count:
                pltpu.sync_copy(zc_v.at[pl.ds(0, 8)],
                                cnt_sh.at[pl.ds(NH, 8)])
        if with_count:
            one0 = jnp.where(
                lax.iota(jnp.int32, 16) == 0, 1.0, 0.0).astype(jnp.float32)

            def co(i, _):
                cmsg_v[i, pl.ds(0, 16)] = one0
                return 0
            lax.fori_loop(0, B, co, 0)
        plsc.subcore_barrier()

        sem_r = (sr0, sr1)
        sem_g = (sg0, sg1)

        def start(j, b):
            pltpu.async_copy(outn.at[s_v.at[j]], rows_v.at[b], sem_r[b])
            pltpu.async_copy(gauss.at[pl.ds(eb + j * B, B)],
                             gs_v.at[b], sem_g[b])

        def wait(j, b):
            pltpu.make_async_copy(
                outn.at[s_v.at[j]], rows_v.at[b], sem_r[b]).wait()
            pltpu.make_async_copy(
                gauss.at[pl.ds(eb + j * B, B)], gs_v.at[b], sem_g[b]).wait()

        def compute_scatter(j, b):
            def edge(i, _):
                accs = [zero16] * nf
                gvec = gs_v[b, i, pl.ds(0, 16)]
                for k in range(K):
                    wk = gvec[k]
                    for f in range(nf):
                        accs[f] = accs[f] + wk * rows_v[
                            b, i, pl.ds(k * fp + 16 * f, 16)]
                for f in range(nf):
                    msg_v[i, pl.ds(16 * f, 16)] = accs[f]
                return 0
            lax.fori_loop(0, B, edge, 0)
            pltpu.sync_copy(msg_v, agg_sh.at[d_v.at[j]], add=True)
            if with_count:
                pltpu.sync_copy(cmsg_v, cnt_sh.at[d_v.at[j]], add=True)

        start(0, 0)

        def step(jj, _):
            j0 = 2 * jj
            wait(j0, 0)
            start(j0 + 1, 1)
            compute_scatter(j0, 0)
            wait(j0 + 1, 1)

            @pl.when(jj < NB // 2 - 1)
            def _():
                start(j0 + 2, 0)
            compute_scatter(j0 + 1, 1)
            return 0
        lax.fori_loop(0, NB // 2, step, 0)

        plsc.subcore_barrier()
        for i in range(4):
            q = s + 16 * i

            @pl.when(q < NCHUNK)
            def _():
                sl = pl.ds(q * DR, DR)
                osl = pl.ds(c * NH + q * DR, DR)
                pltpu.sync_copy(agg_sh.at[sl], out.at[osl])
                if with_count:
                    pltpu.sync_copy(cnt_sh.at[sl], cnt_out.at[osl])

    return pl.kernel(
        body, out_type=out_type, mesh=mesh, scratch_types=scratch,
        compiler_params=pltpu.CompilerParams(use_tc_tiling_on_sc=False))


def _pad_cols(w, out_w):
    return jnp.pad(w, ((0, 0), (0, out_w - w.shape[1])))


def _prep_g(g, f_out, fp, fin_pad=None):
    f_in = g.shape[0]
    g3 = g.reshape(f_in, K, f_out)
    g3 = jnp.pad(g3, ((0, 0), (0, 0), (0, fp - f_out)))
    g2 = g3.reshape(f_in, K * fp)
    if fin_pad is not None and fin_pad != f_in:
        g2 = jnp.pad(g2, ((0, fin_pad - f_in), (0, 0)))
    return g2


def _prep_root(r, fp, fin_pad=None):
    r2 = _pad_cols(r, fp)
    if fin_pad is not None and fin_pad != r.shape[0]:
        r2 = jnp.pad(r2, ((0, fin_pad - r.shape[0]), (0, 0)))
    return r2


def _prep_vec8(v, fp, row=0):
    out = jnp.zeros((8, fp), jnp.float32)
    return out.at[row, :v.shape[0]].set(v)


def _prep_gb(gamma, beta, fp):
    out = jnp.zeros((8, fp), jnp.float32)
    out = out.at[0, :gamma.shape[0]].set(gamma)
    return out.at[1, :beta.shape[0]].set(beta)


def _gauss_coeffs(params):
    c2 = jnp.zeros((8, 64), jnp.float32)
    c1 = jnp.zeros((8, 64), jnp.float32)
    for l, name in enumerate(['conv1', 'conv2', 'conv3', 'conv4']):
        p = params[name]
        inv = -0.5 / (EPS + p['sigma'] ** 2)
        c2 = c2.at[:D, 16 * l:16 * l + K].set(inv.T)
        c1 = c1.at[:D, 16 * l:16 * l + K].set((-2.0 * inv * p['mu']).T)
        c1 = c1.at[3, 16 * l:16 * l + K].set(
            jnp.sum(inv * p['mu'] ** 2, axis=1))
    return c2, c1


def kernel(x, edge_index, edge_attr, params):
    src = edge_index[0].astype(jnp.int32)
    dst = edge_index[1].astype(jnp.int32)

    hi = dst >= NH
    r1 = jnp.cumsum(hi.astype(jnp.int32))
    r0 = jnp.cumsum(1 - hi.astype(jnp.int32))
    slot = jnp.where(hi, C + r1 - 1, r0 - 1)
    src_p = jnp.zeros((SLOTS,), jnp.int32).at[slot].set(
        src, unique_indices=True)
    dst_p = jnp.full((SLOTS,), NH, jnp.int32).at[slot].set(
        dst - jnp.where(hi, NH, 0), unique_indices=True)
    ea8 = jnp.concatenate(
        [edge_attr, jnp.ones((E, 1), jnp.float32),
         jnp.zeros((E, 4), jnp.float32)], axis=1)
    ea_p = jnp.zeros((SLOTS, 8), jnp.float32).at[slot].set(
        ea8, unique_indices=True)

    src2 = src_p.reshape(SLOTS // B, B)
    dst2 = dst_p.reshape(SLOTS // B, B)

    c2, c1 = _gauss_coeffs(params)
    gs = _gauss_all(ea_p, c2, c1)

    sc_msg = [_make_sc_msg(FP[0], True)] + [
        _make_sc_msg(FP[l], False) for l in (1, 2, 3)]

    p1 = params['conv1']

    outn = _matmul(x, _prep_g(p1['g'], FOUT[0], FP[0]))
    agg, cnt = sc_msg[0](outn, gs[0], src2, dst2)
    h, part, invc = _combine1(agg, cnt, x,
                              _prep_root(p1['root'], FP[0]),
                              _prep_vec8(p1['bias'], FP[0]))

    convs = [params['conv2'], params['conv3'], params['conv4']]
    bns = [params['bn1'], params['bn2'], params['bn3']]
    out = None
    for l in (1, 2, 3):
        p = convs[l - 1]
        gb = _prep_gb(bns[l - 1]['gamma'], bns[l - 1]['beta'], FP[l - 1])
        outn = _matmul(h, _prep_g(p['g'], FOUT[l], FP[l], FP[l - 1]),
                       part, gb)
        agg = sc_msg[l](outn, gs[l], src2, dst2)[0]
        res = _combine(agg, invc, h, part, gb,
                       _prep_root(p['root'], FP[l], FP[l - 1]),
                       _prep_vec8(p['bias'], FP[l]), last=(l == 3))
        if l == 3:
            out = res[0]
        else:
            h, part = res

    return out[:, :FOUT[3]]
